# baseline (speedup 1.0000x reference)
import functools

import jax
import jax.numpy as jnp
from jax.experimental import pallas as pl

_C1 = 2000
_C3 = 1000
_NT = 2000


def _k1(dst_ref, sc_ref, cis_ref, rf_ref, hd_ref):
    step = pl.program_id(0)

    @pl.when(step == 0)
    def _():
        hd_ref[...] = jnp.zeros_like(hd_ref)

    iota64 = jax.lax.broadcasted_iota(jnp.int32, (1, 64), 1)

    def body(i, carry):
        d = dst_ref[i, 0]
        t = sc_ref[i, 0]
        cs = cis_ref[i, 0]
        rfrow = rf_ref[pl.ds(i, 1), :]
        darow = jnp.where(iota64 == t, cs, 0.0) + jnp.where(
            iota64 == 5, 1.0, 0.0)
        upd = jnp.concatenate([rfrow, darow], axis=1)
        hd_ref[pl.ds(d, 1), :] = hd_ref[pl.ds(d, 1), :] + upd
        return carry

    jax.lax.fori_loop(0, dst_ref.shape[0], body, 0)


def _k2(nsteps, hd_ref, f2_ref, f3_ref, w1_ref, w2_ref,
        t1_ref, t2_ref, t3_ref,
        rfe3_ref, x2_ref, g1_ref, g2_ref, g3_ref, s_ref, m_ref):
    j = pl.program_id(0)

    @pl.when(j == 0)
    def _():
        s_ref[...] = jnp.zeros_like(s_ref)
        m_ref[...] = jnp.zeros_like(m_ref)

    deg = hd_ref[:, 69:70]
    h = hd_ref[:, 0:64] / jnp.maximum(deg, 1.0)
    w1 = w1_ref[...]
    w2 = w2_ref[...]
    rfe3 = jnp.dot(h, w1.T, preferred_element_type=jnp.float32)
    rfe3_ref[...] = rfe3
    f2 = f2_ref[...]
    x2_ref[...] = f2 + jnp.dot(h, w2.T, preferred_element_type=jnp.float32)

    for t_ref, g_ref in ((t1_ref, g1_ref), (t2_ref, g2_ref), (t3_ref, g3_ref)):
        g = hd_ref[:, 64:65] * t_ref[0:1, :]
        for q in range(1, 5):
            g = g + hd_ref[:, 64 + q:65 + q] * t_ref[q:q + 1, :]
        g_ref[...] = g

    s_ref[0:1, :] = s_ref[0:1, :] + jnp.sum(h, axis=0, keepdims=True)
    s_ref[1:2, :] = s_ref[1:2, :] + jnp.sum(f2, axis=0, keepdims=True)
    s_ref[2:3, :] = s_ref[2:3, :] + jnp.sum(f3_ref[...], axis=0, keepdims=True)

    @pl.when(j == nsteps - 1)
    def _():
        n_nodes = jnp.float32(nsteps * hd_ref.shape[0])
        sh = s_ref[0:1, :] / n_nodes
        m_ref[0:1, :] = jnp.dot(sh, w1.T, preferred_element_type=jnp.float32)
        m_ref[1:2, :] = s_ref[1:2, :] / n_nodes + jnp.dot(
            sh, w2.T, preferred_element_type=jnp.float32)
        m_ref[2:3, :] = s_ref[2:3, :] / n_nodes


def _k3(dst_ref, src_ref, sc_ref, cis_ref, x_ref, t_ref, o_ref):
    step = pl.program_id(0)

    @pl.when(step == 0)
    def _():
        o_ref[...] = jnp.zeros_like(o_ref)

    def body(i, carry):
        d = dst_ref[i, 0]
        s = src_ref[i, 0]
        t = sc_ref[i, 0]
        cs = cis_ref[i, 0]
        xrow = x_ref[pl.ds(s, 1), :]
        trow = t_ref[pl.ds(t, 1), :]
        o_ref[pl.ds(d, 1), :] = o_ref[pl.ds(d, 1), :] + xrow * trow * cs
        return carry

    jax.lax.fori_loop(0, dst_ref.shape[0], body, 0)


def _k4(o1_ref, o2_ref, o3_ref, g1_ref, g2_ref, g3_ref, ci_ref, m_ref,
        rst_ref, rstf_ref, re_ref, id_ref, ref_ref, idf_ref):
    ci = ci_ref[...]
    m1 = m_ref[0:1, :]
    m2 = m_ref[1:2, :]
    m3 = m_ref[2:3, :]
    rst_ref[...] = o2_ref[...] * ci
    rstf_ref[...] = g2_ref[...] * ci * m2
    re_ref[...] = o1_ref[...] * ci
    id_ref[...] = o3_ref[...] * ci
    ref_ref[...] = g1_ref[...] * ci * m1
    idf_ref[...] = g3_ref[...] * ci * m3


def kernel(edge_index, review_feat, score, ci, feature, W1, W2, T1, T2, T3,
           feature2, feature3):
    del feature
    e = edge_index.shape[1]
    n, d = feature2.shape
    f32 = jnp.float32

    src = edge_index[0].astype(jnp.int32).reshape(e, 1)
    dst = edge_index[1].astype(jnp.int32).reshape(e, 1)
    sc0 = (score.astype(jnp.int32) - 1).reshape(e, 1)
    cif = ci.astype(f32)
    cis = cif[src[:, 0]]
    t1p = jnp.pad(T1.astype(f32), ((0, 3), (0, 0)))
    t2p = jnp.pad(T2.astype(f32), ((0, 3), (0, 0)))
    t3p = jnp.pad(T3.astype(f32), ((0, 3), (0, 0)))

    nt = _NT
    ng = n // nt

    def idx1(c):
        return pl.BlockSpec((c, 1), lambda i: (i, 0))

    hd = pl.pallas_call(
        _k1,
        grid=(e // _C1,),
        in_specs=[idx1(_C1), idx1(_C1), idx1(_C1),
                  pl.BlockSpec((_C1, d), lambda i: (i, 0))],
        out_specs=pl.BlockSpec((n, 128), lambda i: (0, 0)),
        out_shape=jax.ShapeDtypeStruct((n, 128), f32),
    )(dst, sc0, cis, review_feat.astype(f32))

    tile_nd = pl.BlockSpec((nt, d), lambda j: (j, 0))
    w_spec = pl.BlockSpec((d, d), lambda j: (0, 0))
    s_spec = pl.BlockSpec((8, d), lambda j: (0, 0))

    rfe3, x2, g1, g2, g3, _, m = pl.pallas_call(
        functools.partial(_k2, ng),
        grid=(ng,),
        in_specs=[pl.BlockSpec((nt, 128), lambda j: (j, 0)),
                  tile_nd, tile_nd, w_spec, w_spec,
                  s_spec, s_spec, s_spec],
        out_specs=[tile_nd, tile_nd, tile_nd, tile_nd, tile_nd,
                   s_spec, s_spec],
        out_shape=[jax.ShapeDtypeStruct((n, d), f32)] * 5 + [
            jax.ShapeDtypeStruct((8, d), f32)] * 2,
    )(hd, feature2.astype(f32), feature3.astype(f32),
      W1.astype(f32), W2.astype(f32), t1p, t2p, t3p)

    def branch(x_tab, t_tab):
        return pl.pallas_call(
            _k3,
            grid=(e // _C3,),
            in_specs=[idx1(_C3), idx1(_C3), idx1(_C3), idx1(_C3),
                      pl.BlockSpec((n, d), lambda i: (0, 0)),
                      pl.BlockSpec((8, d), lambda i: (0, 0))],
            out_specs=pl.BlockSpec((n, d), lambda i: (0, 0)),
            out_shape=jax.ShapeDtypeStruct((n, d), f32),
        )(dst, src, sc0, cis, x_tab, t_tab)

    o1 = branch(rfe3, t1p)
    o2 = branch(x2, t2p)
    o3 = branch(feature3.astype(f32), t3p)

    tile_n1 = pl.BlockSpec((nt, 1), lambda j: (j, 0))
    rst, rstf, rre, rid, rref, ridf = pl.pallas_call(
        _k4,
        grid=(ng,),
        in_specs=[tile_nd] * 6 + [tile_n1, s_spec],
        out_specs=[tile_nd] * 6,
        out_shape=[jax.ShapeDtypeStruct((n, d), f32)] * 6,
    )(o1, o2, o3, g1, g2, g3, cif, m)

    return (rst, rstf, rre, rid, rref, ridf)


# speedup vs baseline: 1.2050x; 1.2050x over previous
import functools

import jax
import jax.numpy as jnp
from jax.experimental import pallas as pl

_C1 = 2000
_C3 = 1000
_NT = 2000


def _k1(dst_ref, sc_ref, cis_ref, rf_ref, hd_ref):
    step = pl.program_id(0)

    @pl.when(step == 0)
    def _():
        hd_ref[...] = jnp.zeros_like(hd_ref)

    iota64 = jax.lax.broadcasted_iota(jnp.int32, (1, 64), 1)

    def body(i, carry):
        d = dst_ref[i, 0]
        t = sc_ref[i, 0]
        cs = cis_ref[i, 0]
        rfrow = rf_ref[pl.ds(i, 1), :]
        darow = jnp.where(iota64 == t, cs, 0.0) + jnp.where(
            iota64 == 5, 1.0, 0.0)
        upd = jnp.concatenate([rfrow, darow], axis=1)
        hd_ref[pl.ds(d, 1), :] = hd_ref[pl.ds(d, 1), :] + upd
        return carry

    jax.lax.fori_loop(0, dst_ref.shape[0], body, 0)


def _k2(nsteps, hd_ref, f2_ref, f3_ref, w1_ref, w2_ref,
        t1_ref, t2_ref, t3_ref,
        rfe3_ref, x2_ref, g1_ref, g2_ref, g3_ref, s_ref, m_ref):
    j = pl.program_id(0)

    @pl.when(j == 0)
    def _():
        s_ref[...] = jnp.zeros_like(s_ref)
        m_ref[...] = jnp.zeros_like(m_ref)

    deg = hd_ref[:, 69:70]
    h = hd_ref[:, 0:64] / jnp.maximum(deg, 1.0)
    w1 = w1_ref[...]
    w2 = w2_ref[...]
    rfe3 = jnp.dot(h, w1.T, preferred_element_type=jnp.float32)
    rfe3_ref[...] = rfe3
    f2 = f2_ref[...]
    x2_ref[...] = f2 + jnp.dot(h, w2.T, preferred_element_type=jnp.float32)

    for t_ref, g_ref in ((t1_ref, g1_ref), (t2_ref, g2_ref), (t3_ref, g3_ref)):
        g = hd_ref[:, 64:65] * t_ref[0:1, :]
        for q in range(1, 5):
            g = g + hd_ref[:, 64 + q:65 + q] * t_ref[q:q + 1, :]
        g_ref[...] = g

    s_ref[0:1, :] = s_ref[0:1, :] + jnp.sum(h, axis=0, keepdims=True)
    s_ref[1:2, :] = s_ref[1:2, :] + jnp.sum(f2, axis=0, keepdims=True)
    s_ref[2:3, :] = s_ref[2:3, :] + jnp.sum(f3_ref[...], axis=0, keepdims=True)

    @pl.when(j == nsteps - 1)
    def _():
        n_nodes = jnp.float32(nsteps * hd_ref.shape[0])
        sh = s_ref[0:1, :] / n_nodes
        m_ref[0:1, :] = jnp.dot(sh, w1.T, preferred_element_type=jnp.float32)
        m_ref[1:2, :] = s_ref[1:2, :] / n_nodes + jnp.dot(
            sh, w2.T, preferred_element_type=jnp.float32)
        m_ref[2:3, :] = s_ref[2:3, :] / n_nodes


def _k3(dst_ref, src_ref, sc_ref, cis_ref, x_ref, t_ref, o_ref):
    step = pl.program_id(0)

    @pl.when(step == 0)
    def _():
        o_ref[...] = jnp.zeros_like(o_ref)

    def body(i, carry):
        d = dst_ref[i, 0]
        s = src_ref[i, 0]
        t = sc_ref[i, 0]
        cs = cis_ref[i, 0]
        xrow = x_ref[pl.ds(s, 1), :]
        trow = t_ref[pl.ds(t, 1), :]
        o_ref[pl.ds(d, 1), :] = o_ref[pl.ds(d, 1), :] + xrow * trow * cs
        return carry

    jax.lax.fori_loop(0, dst_ref.shape[0], body, 0)


def _k4(o1_ref, o2_ref, o3_ref, g1_ref, g2_ref, g3_ref, ci_ref, m_ref,
        rst_ref, rstf_ref, re_ref, id_ref, ref_ref, idf_ref):
    ci = ci_ref[...]
    m1 = m_ref[0:1, :]
    m2 = m_ref[1:2, :]
    m3 = m_ref[2:3, :]
    rst_ref[...] = o2_ref[...] * ci
    rstf_ref[...] = g2_ref[...] * ci * m2
    re_ref[...] = o1_ref[...] * ci
    id_ref[...] = o3_ref[...] * ci
    ref_ref[...] = g1_ref[...] * ci * m1
    idf_ref[...] = g3_ref[...] * ci * m3


def kernel(edge_index, review_feat, score, ci, feature, W1, W2, T1, T2, T3,
           feature2, feature3):
    del feature
    e = edge_index.shape[1]
    n, d = feature2.shape
    f32 = jnp.float32

    src = edge_index[0].astype(jnp.int32).reshape(e, 1)
    dst = edge_index[1].astype(jnp.int32).reshape(e, 1)
    sc0 = (score.astype(jnp.int32) - 1).reshape(e, 1)
    cif = ci.astype(f32)
    cis = cif[src[:, 0]]
    t1p = jnp.pad(T1.astype(f32), ((0, 3), (0, 0)))
    t2p = jnp.pad(T2.astype(f32), ((0, 3), (0, 0)))
    t3p = jnp.pad(T3.astype(f32), ((0, 3), (0, 0)))

    nt = _NT
    ng = n // nt

    def idx1(c):
        return pl.BlockSpec((c, 1), lambda i: (i, 0))

    hd = pl.pallas_call(
        _k1,
        grid=(e // _C1,),
        in_specs=[idx1(_C1), idx1(_C1), idx1(_C1),
                  pl.BlockSpec((_C1, d), lambda i: (i, 0))],
        out_specs=pl.BlockSpec((n, 128), lambda i: (0, 0)),
        out_shape=jax.ShapeDtypeStruct((n, 128), f32),
    )(dst, sc0, cis, review_feat.astype(f32))

    tile_nd = pl.BlockSpec((nt, d), lambda j: (j, 0))
    w_spec = pl.BlockSpec((d, d), lambda j: (0, 0))
    s_spec = pl.BlockSpec((8, d), lambda j: (0, 0))

    rfe3, x2, g1, g2, g3, _, m = pl.pallas_call(
        functools.partial(_k2, ng),
        grid=(ng,),
        in_specs=[pl.BlockSpec((nt, 128), lambda j: (j, 0)),
                  tile_nd, tile_nd, w_spec, w_spec,
                  s_spec, s_spec, s_spec],
        out_specs=[tile_nd, tile_nd, tile_nd, tile_nd, tile_nd,
                   s_spec, s_spec],
        out_shape=[jax.ShapeDtypeStruct((n, d), f32)] * 5 + [
            jax.ShapeDtypeStruct((8, d), f32)] * 2,
    )(hd, feature2.astype(f32), feature3.astype(f32),
      W1.astype(f32), W2.astype(f32), t1p, t2p, t3p)

    def branch(x_tab, t_tab):
        w = x_tab.shape[1]
        return pl.pallas_call(
            _k3,
            grid=(e // _C3,),
            in_specs=[idx1(_C3), idx1(_C3), idx1(_C3), idx1(_C3),
                      pl.BlockSpec((n, w), lambda i: (0, 0)),
                      pl.BlockSpec((8, w), lambda i: (0, 0))],
            out_specs=pl.BlockSpec((n, w), lambda i: (0, 0)),
            out_shape=jax.ShapeDtypeStruct((n, w), f32),
        )(dst, src, sc0, cis, x_tab, t_tab)

    o12 = branch(jnp.concatenate([rfe3, x2], axis=1),
                 jnp.concatenate([t1p, t2p], axis=1))
    o1 = o12[:, :d]
    o2 = o12[:, d:]
    o3 = branch(feature3.astype(f32), t3p)

    tile_n1 = pl.BlockSpec((nt, 1), lambda j: (j, 0))
    rst, rstf, rre, rid, rref, ridf = pl.pallas_call(
        _k4,
        grid=(ng,),
        in_specs=[tile_nd] * 6 + [tile_n1, s_spec],
        out_specs=[tile_nd] * 6,
        out_shape=[jax.ShapeDtypeStruct((n, d), f32)] * 6,
    )(o1, o2, o3, g1, g2, g3, cif, m)

    return (rst, rstf, rre, rid, rref, ridf)
